# baseline (speedup 1.0000x reference)
import functools

import jax
import jax.numpy as jnp
from jax import lax
from jax.experimental import pallas as pl
from jax.experimental.pallas import tpu as pltpu
from jax.experimental.pallas import tpu_sc as plsc

NUM_NODES = 10000
NUM_EDGES = 320000
FEAT = 128

NC = 2
NS = 16
NW = NC * NS

NPAD = 10240
DSEG = NPAD // NS
APAD = 10112
AROWS = APAD // NS
EDGES_PER_W = NUM_EDGES // NW
CHUNK = 128
NCHUNK = 79
EPAD = NCHUNK * CHUNK
DUMMY_DST = APAD - 1

BM = 2000

_mesh = plsc.VectorSubcoreMesh(core_axis_name="c", subcore_axis_name="s")
_sc_params = pltpu.CompilerParams(needs_layout_passes=False,
                                  internal_scratch_in_bytes=65536)


@functools.partial(
    pl.kernel,
    mesh=_mesh,
    out_type=[jax.ShapeDtypeStruct((NC * NPAD,), jnp.float32),
              jax.ShapeDtypeStruct((NW * NPAD,), jnp.float32)],
    scratch_types=[
        pltpu.VMEM((EDGES_PER_W,), jnp.int32),
        pltpu.VMEM((NPAD,), jnp.float32),
        pltpu.VMEM((NS * DSEG,), jnp.float32),
    ],
    compiler_params=_sc_params,
)
def _sc_degree(dst_hbm, out_hbm, hscr_hbm, dbuf, hist, rbuf):
    cid = lax.axis_index("c")
    sid = lax.axis_index("s")
    wid = cid * NS + sid

    def zero(i, _):
        hist[pl.ds(i * 16, 16)] = jnp.zeros((16,), jnp.float32)
        return 0

    lax.fori_loop(0, NPAD // 16, zero, 0)

    pltpu.sync_copy(dst_hbm.at[pl.ds(wid * EDGES_PER_W, EDGES_PER_W)], dbuf)
    ones = jnp.ones((16,), jnp.float32)

    def body(i, _):
        idxv = dbuf[pl.ds(i * 16, 16)]
        plsc.addupdate_scatter(hist, [idxv], ones)
        return 0

    lax.fori_loop(0, EDGES_PER_W // 16, body, 0)

    pltpu.sync_copy(hist, hscr_hbm.at[pl.ds(wid * NPAD, NPAD)])
    plsc.subcore_barrier()
    cbase = sid * DSEG
    for k in range(NS):
        pltpu.sync_copy(
            hscr_hbm.at[pl.ds((cid * NS + k) * NPAD + cbase, DSEG)],
            rbuf.at[pl.ds(k * DSEG, DSEG)])

    def reduce(j, _):
        acc = jnp.zeros((16,), jnp.float32)
        for k in range(NS):
            acc = acc + rbuf[pl.ds(k * DSEG + j * 16, 16)]
        hist[pl.ds(j * 16, 16)] = acc
        return 0

    lax.fori_loop(0, DSEG // 16, reduce, 0)
    pltpu.sync_copy(hist.at[pl.ds(0, DSEG)],
                    out_hbm.at[pl.ds(cid * NPAD + cbase, DSEG)])


@functools.partial(
    pl.kernel,
    mesh=_mesh,
    out_type=jax.ShapeDtypeStruct((NC, APAD, FEAT), jnp.float32),
    scratch_types=[
        pltpu.VMEM((2, CHUNK), jnp.int32),
        pltpu.VMEM((2, CHUNK), jnp.int32),
        pltpu.VMEM((2, CHUNK, FEAT), jnp.float32),
        pltpu.VMEM_SHARED((APAD, FEAT), jnp.float32),
        pltpu.VMEM((16, FEAT), jnp.float32),
        pltpu.SemaphoreType.DMA,
        pltpu.SemaphoreType.DMA,
    ],
    compiler_params=_sc_params,
)
def _sc_propagate(y_hbm, src_hbm, dst_hbm, out_hbm, sidx, didx, rows, acc,
                  zbuf, sem, ssem):
    cid = lax.axis_index("c")
    sid = lax.axis_index("s")
    wid = cid * NS + sid

    for r in range(16):
        for c in range(FEAT // 16):
            zbuf[r, pl.ds(c * 16, 16)] = jnp.zeros((16,), jnp.float32)

    zbase = sid * AROWS

    def zero(i, _):
        pltpu.sync_copy(zbuf, acc.at[pl.ds(zbase + i * 16, 16), :])
        return 0

    lax.fori_loop(0, AROWS // 16, zero, 0)
    pltpu.sync_copy(zbuf.at[pl.ds(0, 8), :],
                    acc.at[pl.ds(zbase + (AROWS // 16) * 16, 8), :])
    plsc.subcore_barrier()

    pltpu.sync_copy(src_hbm.at[wid, 0], sidx.at[0])
    pltpu.sync_copy(dst_hbm.at[wid, 0], didx.at[0])
    pltpu.async_copy(y_hbm.at[sidx.at[0]], rows.at[0], sem).wait()

    def body(i, _):
        cur = lax.rem(i, 2)
        pltpu.sync_copy(src_hbm.at[wid, i + 1], sidx.at[1 - cur])
        pltpu.sync_copy(dst_hbm.at[wid, i + 1], didx.at[1 - cur])
        gd = pltpu.async_copy(y_hbm.at[sidx.at[1 - cur]], rows.at[1 - cur],
                              sem)
        sd = pltpu.async_copy(rows.at[cur], acc.at[didx.at[cur]], ssem,
                              add=True)
        sd.wait()
        gd.wait()
        return 0

    lax.fori_loop(0, NCHUNK - 1, body, 0)
    pltpu.sync_copy(rows.at[lax.rem(NCHUNK - 1, 2)],
                    acc.at[didx.at[lax.rem(NCHUNK - 1, 2)]], add=True)
    plsc.subcore_barrier()

    pltpu.sync_copy(acc.at[pl.ds(zbase, AROWS), :],
                    out_hbm.at[cid, pl.ds(zbase, AROWS), :])


def _scale_body(x_ref, d_ref, o_ref):
    o_ref[...] = x_ref[...] * d_ref[...]


def _layer1_body(s_ref, y_ref, d_ref, w_ref, b_ref, o_ref):
    p = (s_ref[0] + s_ref[1] + y_ref[...]) * d_ref[...]
    h = jnp.dot(p, w_ref[...], preferred_element_type=jnp.float32)
    h = jnp.maximum(h + b_ref[...], 0.0)
    o_ref[...] = h * d_ref[...]


def _layer23_body(s_ref, y_ref, d_ref, w2_ref, b2_ref, w3_ref, b3_ref,
                  mu_ref, ls_ref):
    p = (s_ref[0] + s_ref[1] + y_ref[...]) * d_ref[...]
    mu_ref[...] = jnp.dot(p, w2_ref[...],
                          preferred_element_type=jnp.float32) + b2_ref[...]
    ls_ref[...] = jnp.dot(p, w3_ref[...],
                          preferred_element_type=jnp.float32) + b3_ref[...]


_row_spec = pl.BlockSpec((BM, FEAT), lambda i: (i, 0))
_d_spec = pl.BlockSpec((BM, 1), lambda i: (i, 0))
_part_spec = pl.BlockSpec((NC, BM, FEAT), lambda i: (0, i, 0))
_w_spec = pl.BlockSpec((FEAT, FEAT), lambda i: (0, 0))
_b_spec = pl.BlockSpec((1, FEAT), lambda i: (0, 0))
_grid = (NUM_NODES // BM,)


def _scale(x, dinv):
    return pl.pallas_call(
        _scale_body,
        grid=_grid,
        in_specs=[_row_spec, _d_spec],
        out_specs=_row_spec,
        out_shape=jax.ShapeDtypeStruct((NUM_NODES, FEAT), jnp.float32),
    )(x, dinv)


def _layer1(s_parts, y0, dinv, W1, b1):
    return pl.pallas_call(
        _layer1_body,
        grid=_grid,
        in_specs=[_part_spec, _row_spec, _d_spec, _w_spec, _b_spec],
        out_specs=_row_spec,
        out_shape=jax.ShapeDtypeStruct((NUM_NODES, FEAT), jnp.float32),
    )(s_parts, y0, dinv, W1, b1)


def _layer23(s_parts, y1, dinv, W2, b2, W3, b3):
    return pl.pallas_call(
        _layer23_body,
        grid=_grid,
        in_specs=[_part_spec, _row_spec, _d_spec, _w_spec, _b_spec,
                  _w_spec, _b_spec],
        out_specs=[_row_spec, _row_spec],
        out_shape=[jax.ShapeDtypeStruct((NUM_NODES, FEAT), jnp.float32),
                   jax.ShapeDtypeStruct((NUM_NODES, FEAT), jnp.float32)],
    )(s_parts, y1, dinv, W2, b2, W3, b3)


def kernel(x, edge_index, W1, b1, W2, b2, W3, b3):
    src = edge_index[0].astype(jnp.int32)
    dst = edge_index[1].astype(jnp.int32)
    src2 = src.reshape(NW, EDGES_PER_W)
    dst2 = dst.reshape(NW, EDGES_PER_W)
    pad = ((0, 0), (0, EPAD - EDGES_PER_W))
    src3 = jnp.pad(src2, pad).reshape(NW, NCHUNK, CHUNK)
    dst3 = jnp.pad(dst2, pad,
                   constant_values=DUMMY_DST).reshape(NW, NCHUNK, CHUNK)

    deg_parts, _unused_hist_scratch = _sc_degree(dst)
    deg = (deg_parts[:NUM_NODES]
           + deg_parts[NPAD:NPAD + NUM_NODES] + 1.0)
    dinv = lax.rsqrt(deg).reshape(NUM_NODES, 1)

    y0 = _scale(x, dinv)
    s0 = _sc_propagate(y0, src3, dst3)
    y1 = _layer1(s0, y0, dinv, W1, b1.reshape(1, FEAT))
    s1 = _sc_propagate(y1, src3, dst3)
    mu, logstd = _layer23(s1, y1, dinv, W2, b2.reshape(1, FEAT),
                          W3, b3.reshape(1, FEAT))
    return (mu, logstd)


# speedup vs baseline: 1.1412x; 1.1412x over previous
import functools

import jax
import jax.numpy as jnp
from jax import lax
from jax.experimental import pallas as pl
from jax.experimental.pallas import tpu as pltpu
from jax.experimental.pallas import tpu_sc as plsc

NUM_NODES = 10000
NUM_EDGES = 320000
FEAT = 128

NC = 2
NS = 16
NW = NC * NS

NPAD = 10240
ROWS_PER_TILE = NPAD // NS
EDGES_PER_W = NUM_EDGES // NW
CHUNK = 128
NCHUNK = 79
EPAD = NCHUNK * CHUNK
DUMMY_DST = NPAD - 1

BM = 2000

_mesh = plsc.VectorSubcoreMesh(core_axis_name="c", subcore_axis_name="s")
_sc_params = pltpu.CompilerParams(needs_layout_passes=False)


@functools.partial(
    pl.kernel,
    mesh=_mesh,
    out_type=jax.ShapeDtypeStruct((NC * NPAD,), jnp.float32),
    scratch_types=[
        pltpu.VMEM((EDGES_PER_W,), jnp.int32),
        pltpu.VMEM((NPAD,), jnp.float32),
        pltpu.VMEM_SHARED((NS * (NPAD // 2),), jnp.float32),
        pltpu.VMEM((NS * (NPAD // 2 // NS),), jnp.float32),
    ],
    compiler_params=_sc_params,
)
def _sc_degree(dst_hbm, out_hbm, dbuf, hist, shist, rbuf):
    cid = lax.axis_index("c")
    sid = lax.axis_index("s")
    wid = cid * NS + sid

    def zero(i, _):
        hist[pl.ds(i * 16, 16)] = jnp.zeros((16,), jnp.float32)
        return 0

    lax.fori_loop(0, NPAD // 16, zero, 0)

    pltpu.sync_copy(dst_hbm.at[pl.ds(wid * EDGES_PER_W, EDGES_PER_W)], dbuf)
    ones = jnp.ones((16,), jnp.float32)

    def body(i, _):
        idxv = dbuf[pl.ds(i * 16, 16)]
        plsc.addupdate_scatter(hist, [idxv], ones)
        return 0

    lax.fori_loop(0, EDGES_PER_W // 16, body, 0)

    half_n = NPAD // 2
    seg = half_n // NS
    for h in range(2):
        pltpu.sync_copy(hist.at[pl.ds(h * half_n, half_n)],
                        shist.at[pl.ds(sid * half_n, half_n)])
        plsc.subcore_barrier()
        cbase = sid * seg
        for k in range(NS):
            pltpu.sync_copy(shist.at[pl.ds(k * half_n + cbase, seg)],
                            rbuf.at[pl.ds(k * seg, seg)])

        def reduce(j, _):
            acc = jnp.zeros((16,), jnp.float32)
            for k in range(NS):
                acc = acc + rbuf[pl.ds(k * seg + j * 16, 16)]
            hist[pl.ds(j * 16, 16)] = acc
            return 0

        lax.fori_loop(0, seg // 16, reduce, 0)
        pltpu.sync_copy(hist.at[pl.ds(0, seg)],
                        out_hbm.at[pl.ds(cid * NPAD + h * half_n + cbase,
                                         seg)])
        plsc.subcore_barrier()


@functools.partial(
    pl.kernel,
    mesh=_mesh,
    out_type=jax.ShapeDtypeStruct((NC, NPAD, FEAT), jnp.float32),
    scratch_types=[
        pltpu.VMEM((2, CHUNK), jnp.int32),
        pltpu.VMEM((2, CHUNK), jnp.int32),
        pltpu.VMEM((2, CHUNK, FEAT), jnp.float32),
        pltpu.VMEM_SHARED((NPAD, FEAT), jnp.float32),
        pltpu.VMEM((16, FEAT), jnp.float32),
        pltpu.SemaphoreType.DMA,
        pltpu.SemaphoreType.DMA,
    ],
    compiler_params=_sc_params,
)
def _sc_propagate(y_hbm, src_hbm, dst_hbm, out_hbm, sidx, didx, rows, acc,
                  zbuf, sem, ssem):
    cid = lax.axis_index("c")
    sid = lax.axis_index("s")
    wid = cid * NS + sid

    for r in range(16):
        for c in range(FEAT // 16):
            zbuf[r, pl.ds(c * 16, 16)] = jnp.zeros((16,), jnp.float32)

    zbase = sid * ROWS_PER_TILE

    def zero(i, _):
        pltpu.sync_copy(zbuf, acc.at[pl.ds(zbase + i * 16, 16), :])
        return 0

    lax.fori_loop(0, ROWS_PER_TILE // 16, zero, 0)
    plsc.subcore_barrier()

    G = 2
    for b in range(G):
        pltpu.async_copy(src_hbm.at[wid, b], sidx.at[b], ssem).wait()
        pltpu.async_copy(dst_hbm.at[wid, b], didx.at[b], ssem).wait()
        pltpu.async_copy(y_hbm.at[sidx.at[b]], rows.at[b], sem)

    def body(i, _):
        cur = lax.rem(i, G)
        pltpu.make_async_copy(y_hbm.at[pl.ds(0, CHUNK)], rows.at[cur],
                              sem).wait()
        pltpu.async_copy(rows.at[cur], acc.at[didx.at[cur]], ssem,
                         add=True).wait()
        pltpu.async_copy(src_hbm.at[wid, i + G], sidx.at[cur], ssem).wait()
        pltpu.async_copy(dst_hbm.at[wid, i + G], didx.at[cur], ssem).wait()
        pltpu.async_copy(y_hbm.at[sidx.at[cur]], rows.at[cur], sem)
        return 0

    lax.fori_loop(0, NCHUNK - G, body, 0)

    def tail(i, _):
        cur = lax.rem(i, G)
        pltpu.make_async_copy(y_hbm.at[pl.ds(0, CHUNK)], rows.at[cur],
                              sem).wait()
        pltpu.async_copy(rows.at[cur], acc.at[didx.at[cur]], ssem,
                         add=True).wait()
        return 0

    lax.fori_loop(NCHUNK - G, NCHUNK, tail, 0)
    plsc.subcore_barrier()

    pltpu.sync_copy(acc.at[pl.ds(zbase, ROWS_PER_TILE), :],
                    out_hbm.at[cid, pl.ds(zbase, ROWS_PER_TILE), :])


def _scale_body(x_ref, d_ref, o_ref):
    o_ref[...] = x_ref[...] * d_ref[...]


def _layer1_body(s_ref, y_ref, d_ref, w_ref, b_ref, o_ref):
    p = (s_ref[0] + s_ref[1] + y_ref[...]) * d_ref[...]
    h = jnp.dot(p, w_ref[...], preferred_element_type=jnp.float32)
    h = jnp.maximum(h + b_ref[...], 0.0)
    o_ref[...] = h * d_ref[...]


def _layer23_body(s_ref, y_ref, d_ref, w2_ref, b2_ref, w3_ref, b3_ref,
                  mu_ref, ls_ref):
    p = (s_ref[0] + s_ref[1] + y_ref[...]) * d_ref[...]
    mu_ref[...] = jnp.dot(p, w2_ref[...],
                          preferred_element_type=jnp.float32) + b2_ref[...]
    ls_ref[...] = jnp.dot(p, w3_ref[...],
                          preferred_element_type=jnp.float32) + b3_ref[...]


_row_spec = pl.BlockSpec((BM, FEAT), lambda i: (i, 0))
_d_spec = pl.BlockSpec((BM, 1), lambda i: (i, 0))
_part_spec = pl.BlockSpec((NC, BM, FEAT), lambda i: (0, i, 0))
_w_spec = pl.BlockSpec((FEAT, FEAT), lambda i: (0, 0))
_b_spec = pl.BlockSpec((1, FEAT), lambda i: (0, 0))
_grid = (NUM_NODES // BM,)


def _scale(x, dinv):
    return pl.pallas_call(
        _scale_body,
        grid=_grid,
        in_specs=[_row_spec, _d_spec],
        out_specs=_row_spec,
        out_shape=jax.ShapeDtypeStruct((NUM_NODES, FEAT), jnp.float32),
    )(x, dinv)


def _layer1(s_parts, y0, dinv, W1, b1):
    return pl.pallas_call(
        _layer1_body,
        grid=_grid,
        in_specs=[_part_spec, _row_spec, _d_spec, _w_spec, _b_spec],
        out_specs=_row_spec,
        out_shape=jax.ShapeDtypeStruct((NUM_NODES, FEAT), jnp.float32),
    )(s_parts, y0, dinv, W1, b1)


def _layer23(s_parts, y1, dinv, W2, b2, W3, b3):
    return pl.pallas_call(
        _layer23_body,
        grid=_grid,
        in_specs=[_part_spec, _row_spec, _d_spec, _w_spec, _b_spec,
                  _w_spec, _b_spec],
        out_specs=[_row_spec, _row_spec],
        out_shape=[jax.ShapeDtypeStruct((NUM_NODES, FEAT), jnp.float32),
                   jax.ShapeDtypeStruct((NUM_NODES, FEAT), jnp.float32)],
    )(s_parts, y1, dinv, W2, b2, W3, b3)


def kernel(x, edge_index, W1, b1, W2, b2, W3, b3):
    src = edge_index[0].astype(jnp.int32)
    dst = edge_index[1].astype(jnp.int32)
    src2 = src.reshape(NW, EDGES_PER_W)
    dst2 = dst.reshape(NW, EDGES_PER_W)
    pad = ((0, 0), (0, EPAD - EDGES_PER_W))
    src3 = jnp.pad(src2, pad).reshape(NW, NCHUNK, CHUNK)
    dst3 = jnp.pad(dst2, pad,
                   constant_values=DUMMY_DST).reshape(NW, NCHUNK, CHUNK)

    deg_parts = _sc_degree(dst)
    deg = (deg_parts[:NUM_NODES]
           + deg_parts[NPAD:NPAD + NUM_NODES] + 1.0)
    dinv = lax.rsqrt(deg).reshape(NUM_NODES, 1)

    y0 = _scale(x, dinv)
    s0 = _sc_propagate(y0, src3, dst3)
    y1 = _layer1(s0, y0, dinv, W1, b1.reshape(1, FEAT))
    s1 = _sc_propagate(y1, src3, dst3)
    mu, logstd = _layer23(s1, y1, dinv, W2, b2.reshape(1, FEAT),
                          W3, b3.reshape(1, FEAT))
    return (mu, logstd)


# speedup vs baseline: 1.4354x; 1.2578x over previous
import functools

import jax
import jax.numpy as jnp
from jax import lax
from jax.experimental import pallas as pl
from jax.experimental.pallas import tpu as pltpu
from jax.experimental.pallas import tpu_sc as plsc

NUM_NODES = 10000
NUM_EDGES = 320000
FEAT = 128

NC = 2
NS = 16
NW = NC * NS

NPAD = 10240
EDGES_PER_W = NUM_EDGES // NW
CHUNK = 112
NCHUNK = 90
EPAD = NCHUNK * CHUNK
APAD = 10112
AROWS = APAD // NS
DUMMY_DST = APAD - 1

BM = 2000

_mesh = plsc.VectorSubcoreMesh(core_axis_name="c", subcore_axis_name="s")
_sc_params = pltpu.CompilerParams(needs_layout_passes=False)


@functools.partial(
    pl.kernel,
    mesh=_mesh,
    out_type=jax.ShapeDtypeStruct((NC * NPAD,), jnp.float32),
    scratch_types=[
        pltpu.VMEM((EDGES_PER_W,), jnp.int32),
        pltpu.VMEM((NPAD,), jnp.float32),
        pltpu.VMEM_SHARED((NS * (NPAD // 2),), jnp.float32),
        pltpu.VMEM((NS * (NPAD // 2 // NS),), jnp.float32),
    ],
    compiler_params=_sc_params,
)
def _sc_degree(dst_hbm, out_hbm, dbuf, hist, shist, rbuf):
    cid = lax.axis_index("c")
    sid = lax.axis_index("s")
    wid = cid * NS + sid

    def zero(i, _):
        hist[pl.ds(i * 16, 16)] = jnp.zeros((16,), jnp.float32)
        return 0

    lax.fori_loop(0, NPAD // 16, zero, 0)

    pltpu.sync_copy(dst_hbm.at[pl.ds(wid * EDGES_PER_W, EDGES_PER_W)], dbuf)
    ones = jnp.ones((16,), jnp.float32)

    def body(i, _):
        idxv = dbuf[pl.ds(i * 16, 16)]
        plsc.addupdate_scatter(hist, [idxv], ones)
        return 0

    lax.fori_loop(0, EDGES_PER_W // 16, body, 0)

    half_n = NPAD // 2
    seg = half_n // NS
    for h in range(2):
        pltpu.sync_copy(hist.at[pl.ds(h * half_n, half_n)],
                        shist.at[pl.ds(sid * half_n, half_n)])
        plsc.subcore_barrier()
        cbase = sid * seg
        for k in range(NS):
            pltpu.sync_copy(shist.at[pl.ds(k * half_n + cbase, seg)],
                            rbuf.at[pl.ds(k * seg, seg)])

        def reduce(j, _):
            acc = jnp.zeros((16,), jnp.float32)
            for k in range(NS):
                acc = acc + rbuf[pl.ds(k * seg + j * 16, 16)]
            hist[pl.ds(j * 16, 16)] = acc
            return 0

        lax.fori_loop(0, seg // 16, reduce, 0)
        pltpu.sync_copy(hist.at[pl.ds(0, seg)],
                        out_hbm.at[pl.ds(cid * NPAD + h * half_n + cbase,
                                         seg)])
        plsc.subcore_barrier()


@functools.partial(
    pl.kernel,
    mesh=_mesh,
    out_type=jax.ShapeDtypeStruct((NC, APAD, FEAT), jnp.float32),
    scratch_types=[
        pltpu.VMEM((3, 2, CHUNK), jnp.int32),
        pltpu.VMEM((3, CHUNK, FEAT), jnp.float32),
        pltpu.VMEM_SHARED((APAD, FEAT), jnp.float32),
        pltpu.VMEM((16, FEAT), jnp.float32),
        pltpu.SemaphoreType.DMA,
        pltpu.SemaphoreType.DMA,
    ],
    compiler_params=_sc_params,
)
def _sc_propagate(y_hbm, ei_hbm, out_hbm, eidx, rows, acc, zbuf, sem, ssem):
    cid = lax.axis_index("c")
    sid = lax.axis_index("s")
    wid = cid * NS + sid

    for r in range(16):
        for c in range(FEAT // 16):
            zbuf[r, pl.ds(c * 16, 16)] = jnp.zeros((16,), jnp.float32)

    zbase = sid * AROWS

    def zero(i, _):
        pltpu.sync_copy(zbuf, acc.at[pl.ds(zbase + i * 16, 16), :])
        return 0

    lax.fori_loop(0, AROWS // 16, zero, 0)
    pltpu.sync_copy(zbuf.at[pl.ds(0, 8), :],
                    acc.at[pl.ds(zbase + (AROWS // 16) * 16, 8), :])
    plsc.subcore_barrier()

    G = 3
    for b in range(G):
        pltpu.async_copy(ei_hbm.at[wid, b], eidx.at[b], ssem).wait()
        pltpu.async_copy(y_hbm.at[eidx.at[b, 0]], rows.at[b], sem)

    def body(i, _):
        cur = lax.rem(i, G)
        pltpu.make_async_copy(y_hbm.at[pl.ds(0, CHUNK)], rows.at[cur],
                              sem).wait()
        pltpu.async_copy(rows.at[cur], acc.at[eidx.at[cur, 1]], ssem,
                         add=True).wait()
        pltpu.async_copy(ei_hbm.at[wid, i + G], eidx.at[cur], ssem).wait()
        pltpu.async_copy(y_hbm.at[eidx.at[cur, 0]], rows.at[cur], sem)
        return 0

    lax.fori_loop(0, NCHUNK - G, body, 0)

    def tail(i, _):
        cur = lax.rem(i, G)
        pltpu.make_async_copy(y_hbm.at[pl.ds(0, CHUNK)], rows.at[cur],
                              sem).wait()
        pltpu.async_copy(rows.at[cur], acc.at[eidx.at[cur, 1]], ssem,
                         add=True).wait()
        return 0

    lax.fori_loop(NCHUNK - G, NCHUNK, tail, 0)
    plsc.subcore_barrier()

    pltpu.sync_copy(acc.at[pl.ds(zbase, AROWS), :],
                    out_hbm.at[cid, pl.ds(zbase, AROWS), :])


def _scale_body(x_ref, d_ref, o_ref):
    o_ref[...] = x_ref[...] * d_ref[...]


def _layer1_body(s_ref, y_ref, d_ref, w_ref, b_ref, o_ref):
    p = (s_ref[0] + s_ref[1] + y_ref[...]) * d_ref[...]
    h = jnp.dot(p, w_ref[...], preferred_element_type=jnp.float32)
    h = jnp.maximum(h + b_ref[...], 0.0)
    o_ref[...] = h * d_ref[...]


def _layer23_body(s_ref, y_ref, d_ref, w2_ref, b2_ref, w3_ref, b3_ref,
                  mu_ref, ls_ref):
    p = (s_ref[0] + s_ref[1] + y_ref[...]) * d_ref[...]
    mu_ref[...] = jnp.dot(p, w2_ref[...],
                          preferred_element_type=jnp.float32) + b2_ref[...]
    ls_ref[...] = jnp.dot(p, w3_ref[...],
                          preferred_element_type=jnp.float32) + b3_ref[...]


_row_spec = pl.BlockSpec((BM, FEAT), lambda i: (i, 0))
_d_spec = pl.BlockSpec((BM, 1), lambda i: (i, 0))
_part_spec = pl.BlockSpec((NC, BM, FEAT), lambda i: (0, i, 0))
_w_spec = pl.BlockSpec((FEAT, FEAT), lambda i: (0, 0))
_b_spec = pl.BlockSpec((1, FEAT), lambda i: (0, 0))
_grid = (NUM_NODES // BM,)


def _scale(x, dinv):
    return pl.pallas_call(
        _scale_body,
        grid=_grid,
        in_specs=[_row_spec, _d_spec],
        out_specs=_row_spec,
        out_shape=jax.ShapeDtypeStruct((NUM_NODES, FEAT), jnp.float32),
    )(x, dinv)


def _layer1(s_parts, y0, dinv, W1, b1):
    return pl.pallas_call(
        _layer1_body,
        grid=_grid,
        in_specs=[_part_spec, _row_spec, _d_spec, _w_spec, _b_spec],
        out_specs=_row_spec,
        out_shape=jax.ShapeDtypeStruct((NUM_NODES, FEAT), jnp.float32),
    )(s_parts, y0, dinv, W1, b1)


def _layer23(s_parts, y1, dinv, W2, b2, W3, b3):
    return pl.pallas_call(
        _layer23_body,
        grid=_grid,
        in_specs=[_part_spec, _row_spec, _d_spec, _w_spec, _b_spec,
                  _w_spec, _b_spec],
        out_specs=[_row_spec, _row_spec],
        out_shape=[jax.ShapeDtypeStruct((NUM_NODES, FEAT), jnp.float32),
                   jax.ShapeDtypeStruct((NUM_NODES, FEAT), jnp.float32)],
    )(s_parts, y1, dinv, W2, b2, W3, b3)


def kernel(x, edge_index, W1, b1, W2, b2, W3, b3):
    src = edge_index[0].astype(jnp.int32)
    dst = edge_index[1].astype(jnp.int32)
    src2 = src.reshape(NW, EDGES_PER_W)
    dst2 = dst.reshape(NW, EDGES_PER_W)
    pad = ((0, 0), (0, EPAD - EDGES_PER_W))
    src3 = jnp.pad(src2, pad).reshape(NW, NCHUNK, CHUNK)
    dst3 = jnp.pad(dst2, pad,
                   constant_values=DUMMY_DST).reshape(NW, NCHUNK, CHUNK)
    ei4 = jnp.stack([src3, dst3], axis=2)

    deg_parts = _sc_degree(dst)
    deg = (deg_parts[:NUM_NODES]
           + deg_parts[NPAD:NPAD + NUM_NODES] + 1.0)
    dinv = lax.rsqrt(deg).reshape(NUM_NODES, 1)

    y0 = _scale(x, dinv)
    s0 = _sc_propagate(y0, ei4)
    y1 = _layer1(s0, y0, dinv, W1, b1.reshape(1, FEAT))
    s1 = _sc_propagate(y1, ei4)
    mu, logstd = _layer23(s1, y1, dinv, W2, b2.reshape(1, FEAT),
                          W3, b3.reshape(1, FEAT))
    return (mu, logstd)


# speedup vs baseline: 1.7697x; 1.2329x over previous
import functools

import jax
import jax.numpy as jnp
from jax import lax
from jax.experimental import pallas as pl
from jax.experimental.pallas import tpu as pltpu
from jax.experimental.pallas import tpu_sc as plsc

NUM_NODES = 10000
NUM_EDGES = 320000
FEAT = 128

NC = 2
NS = 16
NW = NC * NS

NPAD = 10240
EDGES_PER_W = NUM_EDGES // NW
CHUNK = 88
NCHUNK = 114
EPAD = NCHUNK * CHUNK
APAD = 10112
AROWS = APAD // NS
DUMMY_DST = APAD - 1

BM = 2000

_mesh = plsc.VectorSubcoreMesh(core_axis_name="c", subcore_axis_name="s")
_sc_params = pltpu.CompilerParams(needs_layout_passes=False)


@functools.partial(
    pl.kernel,
    mesh=_mesh,
    out_type=jax.ShapeDtypeStruct((NC * NPAD,), jnp.float32),
    scratch_types=[
        pltpu.VMEM((EDGES_PER_W,), jnp.int32),
        pltpu.VMEM((NPAD,), jnp.float32),
        pltpu.VMEM_SHARED((NS * (NPAD // 2),), jnp.float32),
        pltpu.VMEM((NS * (NPAD // 2 // NS),), jnp.float32),
    ],
    compiler_params=_sc_params,
)
def _sc_degree(dst_hbm, out_hbm, dbuf, hist, shist, rbuf):
    cid = lax.axis_index("c")
    sid = lax.axis_index("s")
    wid = cid * NS + sid

    def zero(i, _):
        hist[pl.ds(i * 16, 16)] = jnp.zeros((16,), jnp.float32)
        return 0

    lax.fori_loop(0, NPAD // 16, zero, 0)

    pltpu.sync_copy(dst_hbm.at[pl.ds(wid * EDGES_PER_W, EDGES_PER_W)], dbuf)
    ones = jnp.ones((16,), jnp.float32)

    def body(i, _):
        idxv = dbuf[pl.ds(i * 16, 16)]
        plsc.addupdate_scatter(hist, [idxv], ones)
        return 0

    lax.fori_loop(0, EDGES_PER_W // 16, body, 0)

    half_n = NPAD // 2
    seg = half_n // NS
    for h in range(2):
        pltpu.sync_copy(hist.at[pl.ds(h * half_n, half_n)],
                        shist.at[pl.ds(sid * half_n, half_n)])
        plsc.subcore_barrier()
        cbase = sid * seg
        for k in range(NS):
            pltpu.sync_copy(shist.at[pl.ds(k * half_n + cbase, seg)],
                            rbuf.at[pl.ds(k * seg, seg)])

        def reduce(j, _):
            acc = jnp.zeros((16,), jnp.float32)
            for k in range(NS):
                acc = acc + rbuf[pl.ds(k * seg + j * 16, 16)]
            hist[pl.ds(j * 16, 16)] = acc
            return 0

        lax.fori_loop(0, seg // 16, reduce, 0)
        pltpu.sync_copy(hist.at[pl.ds(0, seg)],
                        out_hbm.at[pl.ds(cid * NPAD + h * half_n + cbase,
                                         seg)])
        plsc.subcore_barrier()


@functools.partial(
    pl.kernel,
    mesh=_mesh,
    out_type=jax.ShapeDtypeStruct((NC, APAD, FEAT), jnp.float32),
    scratch_types=[
        pltpu.VMEM((4, 2, CHUNK), jnp.int32),
        pltpu.VMEM((4, CHUNK, FEAT), jnp.float32),
        pltpu.VMEM_SHARED((APAD, FEAT), jnp.float32),
        pltpu.VMEM((16, FEAT), jnp.float32),
        pltpu.SemaphoreType.DMA,
        pltpu.SemaphoreType.DMA,
    ],
    compiler_params=_sc_params,
)
def _sc_propagate(y_hbm, ei_hbm, out_hbm, eidx, rows, acc, zbuf, sem, ssem):
    cid = lax.axis_index("c")
    sid = lax.axis_index("s")
    wid = cid * NS + sid

    for r in range(16):
        for c in range(FEAT // 16):
            zbuf[r, pl.ds(c * 16, 16)] = jnp.zeros((16,), jnp.float32)

    zbase = sid * AROWS

    def zero(i, _):
        pltpu.sync_copy(zbuf, acc.at[pl.ds(zbase + i * 16, 16), :])
        return 0

    lax.fori_loop(0, AROWS // 16, zero, 0)
    pltpu.sync_copy(zbuf.at[pl.ds(0, 8), :],
                    acc.at[pl.ds(zbase + (AROWS // 16) * 16, 8), :])
    plsc.subcore_barrier()

    G = 4
    for b in range(G):
        pltpu.async_copy(ei_hbm.at[wid, b], eidx.at[b], ssem).wait()
        pltpu.async_copy(y_hbm.at[eidx.at[b, 0]], rows.at[b], sem)

    def body(i, _):
        cur = lax.rem(i, G)
        pltpu.make_async_copy(y_hbm.at[pl.ds(0, CHUNK)], rows.at[cur],
                              sem).wait()
        pltpu.async_copy(rows.at[cur], acc.at[eidx.at[cur, 1]], ssem,
                         add=True).wait()
        pltpu.async_copy(ei_hbm.at[wid, i + G], eidx.at[cur], ssem).wait()
        pltpu.async_copy(y_hbm.at[eidx.at[cur, 0]], rows.at[cur], sem)
        return 0

    lax.fori_loop(0, NCHUNK - G, body, 0)

    def tail(i, _):
        cur = lax.rem(i, G)
        pltpu.make_async_copy(y_hbm.at[pl.ds(0, CHUNK)], rows.at[cur],
                              sem).wait()
        pltpu.async_copy(rows.at[cur], acc.at[eidx.at[cur, 1]], ssem,
                         add=True).wait()
        return 0

    lax.fori_loop(NCHUNK - G, NCHUNK, tail, 0)
    plsc.subcore_barrier()

    pltpu.sync_copy(acc.at[pl.ds(zbase, AROWS), :],
                    out_hbm.at[cid, pl.ds(zbase, AROWS), :])


def _scale_body(x_ref, d_ref, o_ref):
    o_ref[...] = x_ref[...] * d_ref[...]


def _layer1_body(s_ref, y_ref, d_ref, w_ref, b_ref, o_ref):
    p = (s_ref[0] + s_ref[1] + y_ref[...]) * d_ref[...]
    h = jnp.dot(p, w_ref[...], preferred_element_type=jnp.float32)
    h = jnp.maximum(h + b_ref[...], 0.0)
    o_ref[...] = h * d_ref[...]


def _layer23_body(s_ref, y_ref, d_ref, w2_ref, b2_ref, w3_ref, b3_ref,
                  mu_ref, ls_ref):
    p = (s_ref[0] + s_ref[1] + y_ref[...]) * d_ref[...]
    mu_ref[...] = jnp.dot(p, w2_ref[...],
                          preferred_element_type=jnp.float32) + b2_ref[...]
    ls_ref[...] = jnp.dot(p, w3_ref[...],
                          preferred_element_type=jnp.float32) + b3_ref[...]


_row_spec = pl.BlockSpec((BM, FEAT), lambda i: (i, 0))
_d_spec = pl.BlockSpec((BM, 1), lambda i: (i, 0))
_part_spec = pl.BlockSpec((NC, BM, FEAT), lambda i: (0, i, 0))
_w_spec = pl.BlockSpec((FEAT, FEAT), lambda i: (0, 0))
_b_spec = pl.BlockSpec((1, FEAT), lambda i: (0, 0))
_grid = (NUM_NODES // BM,)


def _scale(x, dinv):
    return pl.pallas_call(
        _scale_body,
        grid=_grid,
        in_specs=[_row_spec, _d_spec],
        out_specs=_row_spec,
        out_shape=jax.ShapeDtypeStruct((NUM_NODES, FEAT), jnp.float32),
    )(x, dinv)


def _layer1(s_parts, y0, dinv, W1, b1):
    return pl.pallas_call(
        _layer1_body,
        grid=_grid,
        in_specs=[_part_spec, _row_spec, _d_spec, _w_spec, _b_spec],
        out_specs=_row_spec,
        out_shape=jax.ShapeDtypeStruct((NUM_NODES, FEAT), jnp.float32),
    )(s_parts, y0, dinv, W1, b1)


def _layer23(s_parts, y1, dinv, W2, b2, W3, b3):
    return pl.pallas_call(
        _layer23_body,
        grid=_grid,
        in_specs=[_part_spec, _row_spec, _d_spec, _w_spec, _b_spec,
                  _w_spec, _b_spec],
        out_specs=[_row_spec, _row_spec],
        out_shape=[jax.ShapeDtypeStruct((NUM_NODES, FEAT), jnp.float32),
                   jax.ShapeDtypeStruct((NUM_NODES, FEAT), jnp.float32)],
    )(s_parts, y1, dinv, W2, b2, W3, b3)


def kernel(x, edge_index, W1, b1, W2, b2, W3, b3):
    src = edge_index[0].astype(jnp.int32)
    dst = edge_index[1].astype(jnp.int32)
    src2 = src.reshape(NW, EDGES_PER_W)
    dst2 = dst.reshape(NW, EDGES_PER_W)
    pad = ((0, 0), (0, EPAD - EDGES_PER_W))
    src3 = jnp.pad(src2, pad).reshape(NW, NCHUNK, CHUNK)
    dst3 = jnp.pad(dst2, pad,
                   constant_values=DUMMY_DST).reshape(NW, NCHUNK, CHUNK)
    ei4 = jnp.stack([src3, dst3], axis=2)

    deg_parts = _sc_degree(dst)
    deg = (deg_parts[:NUM_NODES]
           + deg_parts[NPAD:NPAD + NUM_NODES] + 1.0)
    dinv = lax.rsqrt(deg).reshape(NUM_NODES, 1)

    y0 = _scale(x, dinv)
    s0 = _sc_propagate(y0, ei4)
    y1 = _layer1(s0, y0, dinv, W1, b1.reshape(1, FEAT))
    s1 = _sc_propagate(y1, ei4)
    mu, logstd = _layer23(s1, y1, dinv, W2, b2.reshape(1, FEAT),
                          W3, b3.reshape(1, FEAT))
    return (mu, logstd)


# speedup vs baseline: 1.9158x; 1.0825x over previous
import functools

import jax
import jax.numpy as jnp
from jax import lax
from jax.experimental import pallas as pl
from jax.experimental.pallas import tpu as pltpu
from jax.experimental.pallas import tpu_sc as plsc

NUM_NODES = 10000
NUM_EDGES = 320000
FEAT = 128

NC = 2
NS = 16
NW = NC * NS

NPAD = 10240
EDGES_PER_W = NUM_EDGES // NW
CHUNK = 72
NCHUNK = 139
EPAD = NCHUNK * CHUNK
APAD = 10112
AROWS = APAD // NS
DUMMY_DST = APAD - 1

BM = 2000

_mesh = plsc.VectorSubcoreMesh(core_axis_name="c", subcore_axis_name="s")
_sc_params = pltpu.CompilerParams(needs_layout_passes=False)


@functools.partial(
    pl.kernel,
    mesh=_mesh,
    out_type=jax.ShapeDtypeStruct((NC * NPAD,), jnp.float32),
    scratch_types=[
        pltpu.VMEM((EDGES_PER_W,), jnp.int32),
        pltpu.VMEM((NPAD,), jnp.float32),
        pltpu.VMEM_SHARED((NS * (NPAD // 2),), jnp.float32),
        pltpu.VMEM((NS * (NPAD // 2 // NS),), jnp.float32),
    ],
    compiler_params=_sc_params,
)
def _sc_degree(dst_hbm, out_hbm, dbuf, hist, shist, rbuf):
    cid = lax.axis_index("c")
    sid = lax.axis_index("s")
    wid = cid * NS + sid

    def zero(i, _):
        hist[pl.ds(i * 16, 16)] = jnp.zeros((16,), jnp.float32)
        return 0

    lax.fori_loop(0, NPAD // 16, zero, 0)

    pltpu.sync_copy(dst_hbm.at[pl.ds(wid * EDGES_PER_W, EDGES_PER_W)], dbuf)
    ones = jnp.ones((16,), jnp.float32)

    def body(i, _):
        idxv = dbuf[pl.ds(i * 16, 16)]
        plsc.addupdate_scatter(hist, [idxv], ones)
        return 0

    lax.fori_loop(0, EDGES_PER_W // 16, body, 0)

    half_n = NPAD // 2
    seg = half_n // NS
    for h in range(2):
        pltpu.sync_copy(hist.at[pl.ds(h * half_n, half_n)],
                        shist.at[pl.ds(sid * half_n, half_n)])
        plsc.subcore_barrier()
        cbase = sid * seg
        for k in range(NS):
            pltpu.sync_copy(shist.at[pl.ds(k * half_n + cbase, seg)],
                            rbuf.at[pl.ds(k * seg, seg)])

        def reduce(j, _):
            acc = jnp.zeros((16,), jnp.float32)
            for k in range(NS):
                acc = acc + rbuf[pl.ds(k * seg + j * 16, 16)]
            hist[pl.ds(j * 16, 16)] = acc
            return 0

        lax.fori_loop(0, seg // 16, reduce, 0)
        pltpu.sync_copy(hist.at[pl.ds(0, seg)],
                        out_hbm.at[pl.ds(cid * NPAD + h * half_n + cbase,
                                         seg)])
        plsc.subcore_barrier()


@functools.partial(
    pl.kernel,
    mesh=_mesh,
    out_type=jax.ShapeDtypeStruct((NC, APAD, FEAT), jnp.float32),
    scratch_types=[
        pltpu.VMEM((5, 2, CHUNK), jnp.int32),
        pltpu.VMEM((5, CHUNK, FEAT), jnp.float32),
        pltpu.VMEM_SHARED((APAD, FEAT), jnp.float32),
        pltpu.VMEM((16, FEAT), jnp.float32),
        pltpu.SemaphoreType.DMA,
        pltpu.SemaphoreType.DMA,
    ],
    compiler_params=_sc_params,
)
def _sc_propagate(y_hbm, ei_hbm, out_hbm, eidx, rows, acc, zbuf, sem, ssem):
    cid = lax.axis_index("c")
    sid = lax.axis_index("s")
    wid = cid * NS + sid

    for r in range(16):
        for c in range(FEAT // 16):
            zbuf[r, pl.ds(c * 16, 16)] = jnp.zeros((16,), jnp.float32)

    zbase = sid * AROWS

    def zero(i, _):
        pltpu.sync_copy(zbuf, acc.at[pl.ds(zbase + i * 16, 16), :])
        return 0

    lax.fori_loop(0, AROWS // 16, zero, 0)
    pltpu.sync_copy(zbuf.at[pl.ds(0, 8), :],
                    acc.at[pl.ds(zbase + (AROWS // 16) * 16, 8), :])
    plsc.subcore_barrier()

    G = 5
    for b in range(G):
        pltpu.async_copy(ei_hbm.at[wid, b], eidx.at[b], ssem).wait()
        pltpu.async_copy(y_hbm.at[eidx.at[b, 0]], rows.at[b], sem)

    def body(i, _):
        cur = lax.rem(i, G)
        pltpu.make_async_copy(y_hbm.at[pl.ds(0, CHUNK)], rows.at[cur],
                              sem).wait()
        pltpu.async_copy(rows.at[cur], acc.at[eidx.at[cur, 1]], ssem,
                         add=True).wait()
        pltpu.async_copy(ei_hbm.at[wid, i + G], eidx.at[cur], ssem).wait()
        pltpu.async_copy(y_hbm.at[eidx.at[cur, 0]], rows.at[cur], sem)
        return 0

    lax.fori_loop(0, NCHUNK - G, body, 0)

    def tail(i, _):
        cur = lax.rem(i, G)
        pltpu.make_async_copy(y_hbm.at[pl.ds(0, CHUNK)], rows.at[cur],
                              sem).wait()
        pltpu.async_copy(rows.at[cur], acc.at[eidx.at[cur, 1]], ssem,
                         add=True).wait()
        return 0

    lax.fori_loop(NCHUNK - G, NCHUNK, tail, 0)
    plsc.subcore_barrier()

    pltpu.sync_copy(acc.at[pl.ds(zbase, AROWS), :],
                    out_hbm.at[cid, pl.ds(zbase, AROWS), :])


def _scale_body(x_ref, d_ref, o_ref):
    o_ref[...] = x_ref[...] * d_ref[...]


def _layer1_body(s_ref, y_ref, d_ref, w_ref, b_ref, o_ref):
    p = (s_ref[0] + s_ref[1] + y_ref[...]) * d_ref[...]
    h = jnp.dot(p, w_ref[...], preferred_element_type=jnp.float32)
    h = jnp.maximum(h + b_ref[...], 0.0)
    o_ref[...] = h * d_ref[...]


def _layer23_body(s_ref, y_ref, d_ref, w2_ref, b2_ref, w3_ref, b3_ref,
                  mu_ref, ls_ref):
    p = (s_ref[0] + s_ref[1] + y_ref[...]) * d_ref[...]
    mu_ref[...] = jnp.dot(p, w2_ref[...],
                          preferred_element_type=jnp.float32) + b2_ref[...]
    ls_ref[...] = jnp.dot(p, w3_ref[...],
                          preferred_element_type=jnp.float32) + b3_ref[...]


_row_spec = pl.BlockSpec((BM, FEAT), lambda i: (i, 0))
_d_spec = pl.BlockSpec((BM, 1), lambda i: (i, 0))
_part_spec = pl.BlockSpec((NC, BM, FEAT), lambda i: (0, i, 0))
_w_spec = pl.BlockSpec((FEAT, FEAT), lambda i: (0, 0))
_b_spec = pl.BlockSpec((1, FEAT), lambda i: (0, 0))
_grid = (NUM_NODES // BM,)


def _scale(x, dinv):
    return pl.pallas_call(
        _scale_body,
        grid=_grid,
        in_specs=[_row_spec, _d_spec],
        out_specs=_row_spec,
        out_shape=jax.ShapeDtypeStruct((NUM_NODES, FEAT), jnp.float32),
    )(x, dinv)


def _layer1(s_parts, y0, dinv, W1, b1):
    return pl.pallas_call(
        _layer1_body,
        grid=_grid,
        in_specs=[_part_spec, _row_spec, _d_spec, _w_spec, _b_spec],
        out_specs=_row_spec,
        out_shape=jax.ShapeDtypeStruct((NUM_NODES, FEAT), jnp.float32),
    )(s_parts, y0, dinv, W1, b1)


def _layer23(s_parts, y1, dinv, W2, b2, W3, b3):
    return pl.pallas_call(
        _layer23_body,
        grid=_grid,
        in_specs=[_part_spec, _row_spec, _d_spec, _w_spec, _b_spec,
                  _w_spec, _b_spec],
        out_specs=[_row_spec, _row_spec],
        out_shape=[jax.ShapeDtypeStruct((NUM_NODES, FEAT), jnp.float32),
                   jax.ShapeDtypeStruct((NUM_NODES, FEAT), jnp.float32)],
    )(s_parts, y1, dinv, W2, b2, W3, b3)


def kernel(x, edge_index, W1, b1, W2, b2, W3, b3):
    src = edge_index[0].astype(jnp.int32)
    dst = edge_index[1].astype(jnp.int32)
    src2 = src.reshape(NW, EDGES_PER_W)
    dst2 = dst.reshape(NW, EDGES_PER_W)
    pad = ((0, 0), (0, EPAD - EDGES_PER_W))
    src3 = jnp.pad(src2, pad).reshape(NW, NCHUNK, CHUNK)
    dst3 = jnp.pad(dst2, pad,
                   constant_values=DUMMY_DST).reshape(NW, NCHUNK, CHUNK)
    ei4 = jnp.stack([src3, dst3], axis=2)

    deg_parts = _sc_degree(dst)
    deg = (deg_parts[:NUM_NODES]
           + deg_parts[NPAD:NPAD + NUM_NODES] + 1.0)
    dinv = lax.rsqrt(deg).reshape(NUM_NODES, 1)

    y0 = _scale(x, dinv)
    s0 = _sc_propagate(y0, ei4)
    y1 = _layer1(s0, y0, dinv, W1, b1.reshape(1, FEAT))
    s1 = _sc_propagate(y1, ei4)
    mu, logstd = _layer23(s1, y1, dinv, W2, b2.reshape(1, FEAT),
                          W3, b3.reshape(1, FEAT))
    return (mu, logstd)
